# initial kernel scaffold (unmeasured)
import jax
import jax.numpy as jnp
from jax import lax
from jax.experimental import pallas as pl
from jax.experimental.pallas import tpu as pltpu

SCALE = 64 ** -0.5


def _ag_body(k_ref, v_ref, ok_ref, ov_ref, send_sems, recv_sems):
    my_x = lax.axis_index("x")
    my_y = lax.axis_index("y")
    nbr = (1 - my_x, my_y)

    barrier = pltpu.get_barrier_semaphore()
    pl.semaphore_signal(
        barrier, inc=1, device_id=nbr, device_id_type=pl.DeviceIdType.MESH
    )
    pl.semaphore_wait(barrier, 1)

    s = k_ref.shape[1]

    def exchange(lo):
        ok_ref[:, lo:lo + s] = k_ref[...].astype(jnp.bfloat16)
        ov_ref[:, lo:lo + s] = v_ref[...].astype(jnp.bfloat16)
        rk = pltpu.make_async_remote_copy(
            src_ref=ok_ref.at[:, lo:lo + s],
            dst_ref=ok_ref.at[:, lo:lo + s],
            send_sem=send_sems.at[0],
            recv_sem=recv_sems.at[0],
            device_id=nbr,
            device_id_type=pl.DeviceIdType.MESH,
        )
        rv = pltpu.make_async_remote_copy(
            src_ref=ov_ref.at[:, lo:lo + s],
            dst_ref=ov_ref.at[:, lo:lo + s],
            send_sem=send_sems.at[1],
            recv_sem=recv_sems.at[1],
            device_id=nbr,
            device_id_type=pl.DeviceIdType.MESH,
        )
        rk.start()
        rv.start()
        rk.wait()
        rv.wait()

    @pl.when(my_x == 0)
    def _():
        exchange(0)

    @pl.when(my_x == 1)
    def _():
        exchange(s)


def _ag_kv(K3, V3):
    b, s, hd = K3.shape
    out_shape = jax.ShapeDtypeStruct((b, 2 * s, hd), jnp.bfloat16)
    return pl.pallas_call(
        _ag_body,
        out_shape=[out_shape, out_shape],
        in_specs=[pl.BlockSpec(memory_space=pltpu.VMEM)] * 2,
        out_specs=[pl.BlockSpec(memory_space=pltpu.VMEM)] * 2,
        scratch_shapes=[
            pltpu.SemaphoreType.DMA((2,)),
            pltpu.SemaphoreType.DMA((2,)),
        ],
        compiler_params=pltpu.CompilerParams(collective_id=0),
    )(K3, V3)


def _attn_body(q_ref, k_ref, v_ref, o_ref):
    q = q_ref[...].astype(jnp.bfloat16)
    k = k_ref[...]
    v = v_ref[...]
    s = lax.dot_general(
        q, k, (((1,), (1,)), ((), ())), preferred_element_type=jnp.float32
    ) * SCALE
    m = jnp.max(s, axis=1, keepdims=True)
    p = jnp.exp(s - m)
    denom = jnp.sum(p, axis=1, keepdims=True)
    o = lax.dot_general(
        p.astype(jnp.bfloat16), v, (((1,), (0,)), ((), ())),
        preferred_element_type=jnp.float32,
    )
    o_ref[...] = o / denom


def _attn(Q, Kf, Vf):
    b, sq, h, d = Q.shape
    sk = Kf.shape[1]
    return pl.pallas_call(
        _attn_body,
        grid=(b, h),
        in_specs=[
            pl.BlockSpec((None, sq, None, d), lambda i, j: (i, 0, j, 0)),
            pl.BlockSpec((None, sk, None, d), lambda i, j: (i, 0, j, 0)),
            pl.BlockSpec((None, sk, None, d), lambda i, j: (i, 0, j, 0)),
        ],
        out_specs=pl.BlockSpec((None, sq, None, d), lambda i, j: (i, 0, j, 0)),
        out_shape=jax.ShapeDtypeStruct((b, sq, h, d), jnp.float32),
    )(Q, Kf, Vf)


def kernel(Q, K, V):
    b, s, h, d = Q.shape
    Kf3, Vf3 = _ag_kv(K.reshape(b, s, h * d), V.reshape(b, s, h * d))
    Kf = Kf3.reshape(b, 2 * s, h, d)
    Vf = Vf3.reshape(b, 2 * s, h, d)
    return _attn(Q, Kf, Vf)


# baseline (device time: 25394 ns/iter reference)
import jax
import jax.numpy as jnp
from jax import lax
from jax.experimental import pallas as pl
from jax.experimental.pallas import tpu as pltpu

SCALE = 64 ** -0.5


def _ag_body(k_ref, v_ref, ok_ref, ov_ref, send_sems, recv_sems):
    my_x = lax.axis_index("x")
    my_y = lax.axis_index("y")
    nbr = (1 - my_x, my_y)

    barrier = pltpu.get_barrier_semaphore()
    pl.semaphore_signal(
        barrier, inc=1, device_id=nbr, device_id_type=pl.DeviceIdType.MESH
    )
    pl.semaphore_wait(barrier, 1)

    s = k_ref.shape[1]

    def exchange(lo):
        ok_ref[:, lo:lo + s] = k_ref[...].astype(jnp.bfloat16)
        ov_ref[:, lo:lo + s] = v_ref[...].astype(jnp.bfloat16)
        rk = pltpu.make_async_remote_copy(
            src_ref=ok_ref.at[:, lo:lo + s],
            dst_ref=ok_ref.at[:, lo:lo + s],
            send_sem=send_sems.at[0],
            recv_sem=recv_sems.at[0],
            device_id=nbr,
            device_id_type=pl.DeviceIdType.MESH,
        )
        rv = pltpu.make_async_remote_copy(
            src_ref=ov_ref.at[:, lo:lo + s],
            dst_ref=ov_ref.at[:, lo:lo + s],
            send_sem=send_sems.at[1],
            recv_sem=recv_sems.at[1],
            device_id=nbr,
            device_id_type=pl.DeviceIdType.MESH,
        )
        rk.start()
        rv.start()
        rk.wait()
        rv.wait()

    @pl.when(my_x == 0)
    def _():
        exchange(0)

    @pl.when(my_x == 1)
    def _():
        exchange(s)


def _ag_kv(K3, V3):
    b, s, hd = K3.shape
    out_shape = jax.ShapeDtypeStruct((b, 2 * s, hd), jnp.bfloat16)
    return pl.pallas_call(
        _ag_body,
        out_shape=[out_shape, out_shape],
        in_specs=[pl.BlockSpec(memory_space=pltpu.VMEM)] * 2,
        out_specs=[pl.BlockSpec(memory_space=pltpu.VMEM)] * 2,
        scratch_shapes=[
            pltpu.SemaphoreType.DMA((2,)),
            pltpu.SemaphoreType.DMA((2,)),
        ],
        compiler_params=pltpu.CompilerParams(collective_id=0),
    )(K3, V3)


def _make_attn_body(h, d):
    def _attn_body(q_ref, k_ref, v_ref, o_ref):
        q = q_ref[...].astype(jnp.bfloat16)
        k = k_ref[...]
        v = v_ref[...]
        for i in range(h):
            qh = q[:, i * d:(i + 1) * d]
            kh = k[:, i * d:(i + 1) * d]
            vh = v[:, i * d:(i + 1) * d]
            s = lax.dot_general(
                qh, kh, (((1,), (1,)), ((), ())),
                preferred_element_type=jnp.float32,
            ) * SCALE
            m = jnp.max(s, axis=1, keepdims=True)
            p = jnp.exp(s - m)
            denom = jnp.sum(p, axis=1, keepdims=True)
            o = lax.dot_general(
                p.astype(jnp.bfloat16), vh, (((1,), (0,)), ((), ())),
                preferred_element_type=jnp.float32,
            )
            o_ref[:, i * d:(i + 1) * d] = o / denom
    return _attn_body


def _attn(Q3, Kf3, Vf3, h, d):
    b, sq, hd = Q3.shape
    sk = Kf3.shape[1]
    return pl.pallas_call(
        _make_attn_body(h, d),
        grid=(b,),
        in_specs=[
            pl.BlockSpec((None, sq, hd), lambda i: (i, 0, 0)),
            pl.BlockSpec((None, sk, hd), lambda i: (i, 0, 0)),
            pl.BlockSpec((None, sk, hd), lambda i: (i, 0, 0)),
        ],
        out_specs=pl.BlockSpec((None, sq, hd), lambda i: (i, 0, 0)),
        out_shape=jax.ShapeDtypeStruct((b, sq, hd), jnp.float32),
    )(Q3, Kf3, Vf3)


def kernel(Q, K, V):
    b, s, h, d = Q.shape
    Kf3, Vf3 = _ag_kv(K.reshape(b, s, h * d), V.reshape(b, s, h * d))
    out3 = _attn(Q.reshape(b, s, h * d), Kf3, Vf3, h, d)
    return out3.reshape(b, s, h, d)
